# baseline (speedup 1.0000x reference)
import functools

import jax
import jax.numpy as jnp
from jax import lax
from jax.experimental import pallas as pl
from jax.experimental.pallas import tpu as pltpu
from jax.experimental.pallas import tpu_sc as plsc

NC = 2
NS = 16


def _sc_propagate(np_, e, w, nb, k, const_u=False, dtype=jnp.float32):
    npass = max(nb // NC, 1)
    out_sc = NC if nb == 1 else nb
    if nb == 1:
        nchunk = e // k // (NC * NS)
    else:
        nchunk = e // k // NS
    NBUF = 4
    assert nchunk % NBUF == 0
    rpt = np_ // NS

    @functools.partial(
        pl.kernel,
        out_type=jax.ShapeDtypeStruct((out_sc, np_, w), dtype),
        mesh=plsc.VectorSubcoreMesh(core_axis_name="c", subcore_axis_name="s"),
        compiler_params=pltpu.CompilerParams(use_tc_tiling_on_sc=False),
        scratch_types=[
            pltpu.VMEM_SHARED((np_, w), dtype),
            pltpu.VMEM((nchunk, k), jnp.int32),
            pltpu.VMEM((nchunk, k), jnp.int32),
        ] + [pltpu.VMEM((k, w), dtype)] * NBUF
          + [pltpu.SemaphoreType.DMA] * (2 * NBUF),
    )
    def prop(u_hbm, src_hbm, dst_hbm, out_hbm, acc_sh, src_v, dst_v, *rest):
        bufs = rest[:NBUF]
        gsem = rest[NBUF:2 * NBUF]
        ssem = rest[2 * NBUF:]
        c = lax.axis_index("c")
        s = lax.axis_index("s")
        r0 = s * rpt
        if nb == 1:
            dsec = c * NS + s
        else:
            dsec = s
        pltpu.sync_copy(dst_hbm.at[dsec], dst_v)

        def start_g(uj, i, b):
            pltpu.async_copy(uj.at[src_v.at[i]], bufs[b], gsem[b])

        def wait_g(uj, b):
            pltpu.make_async_copy(uj.at[src_v.at[0]], bufs[b], gsem[b]).wait()

        def start_s(i, b):
            pltpu.async_copy(bufs[b], acc_sh.at[dst_v.at[i]], ssem[b], add=True)

        def wait_s(b):
            pltpu.make_async_copy(bufs[b], acc_sh.at[dst_v.at[0]], ssem[b]).wait()

        for p in range(npass):
            if nb == 1:
                j = 0
                ssec = dsec
            else:
                j = p * NC + c
                ssec = s
            uj = u_hbm.at[j]
            if not const_u and p == 0:
                pltpu.sync_copy(src_hbm.at[ssec], src_v)
            pltpu.sync_copy(u_hbm.at[j, pl.ds(r0, rpt), :],
                            acc_sh.at[pl.ds(r0, rpt), :])
            if const_u and p == 0:
                pltpu.sync_copy(u_hbm.at[0, pl.ds(0, k), :], bufs[0])
            plsc.subcore_barrier()

            if const_u:
                def body_c(i, carry):
                    pltpu.sync_copy(bufs[0], acc_sh.at[dst_v.at[i]], add=True)
                    return carry

                lax.fori_loop(0, nchunk, body_c, 0)
            else:
                for b in range(NBUF):
                    start_g(uj, b, b)

                def body(i, carry):
                    ei = NBUF * i
                    for b in range(NBUF):
                        wait_g(uj, b)
                        start_s(ei + b, b)
                    for b in range(NBUF):
                        wait_s(b)
                        start_g(uj, jnp.minimum(ei + b + NBUF, nchunk - 1), b)
                    return carry

                lax.fori_loop(0, nchunk // NBUF, body, 0)
                for b in range(NBUF):
                    wait_g(uj, b)
            plsc.subcore_barrier()
            oj = c if nb == 1 else j
            pltpu.sync_copy(acc_sh.at[pl.ds(r0, rpt), :],
                            out_hbm.at[oj, pl.ds(r0, rpt), :])
            if p + 1 < npass:
                plsc.subcore_barrier()

    return prop


def _sc_layer1(np_, e, k):
    ncw = e // k // NS
    NBUF = 4
    assert ncw % NBUF == 0
    rpt = np_ // NS
    RB = 128
    assert rpt % RB == 0

    @functools.partial(
        pl.kernel,
        out_type=(
            jax.ShapeDtypeStruct((2, np_, 128), jnp.bfloat16),
            jax.ShapeDtypeStruct((np_, 16), jnp.float32),
            jax.ShapeDtypeStruct((2, np_, 128), jnp.bfloat16),
        ),
        mesh=plsc.VectorSubcoreMesh(core_axis_name="c", subcore_axis_name="s"),
        compiler_params=pltpu.CompilerParams(use_tc_tiling_on_sc=False,
                                             needs_layout_passes=False),
        scratch_types=[
            pltpu.VMEM_SHARED((np_, 128), jnp.bfloat16),
            pltpu.VMEM_SHARED((np_, 16), jnp.float32),
            pltpu.VMEM((ncw, k), jnp.int32),
            pltpu.VMEM((ncw, k), jnp.int32),
            pltpu.VMEM((k, 16), jnp.float32),
            pltpu.VMEM((RB, 16), jnp.float32),
            pltpu.VMEM((RB, 16), jnp.float32),
            pltpu.VMEM((RB, 128), jnp.bfloat16),
            pltpu.VMEM((RB, 128), jnp.bfloat16),
        ] + [pltpu.VMEM((k, 128), jnp.bfloat16)] * NBUF
          + [pltpu.SemaphoreType.DMA] * (2 * NBUF),
    )
    def p1(xbf, ones_hbm, src_hbm, dst_hbm, acc1_o, dinv_o, u0_o,
           acc_sh, deg_sh, src_v, dst_v, onesb, degb, dvb, xb, ub, *rest):
        bufs = rest[:NBUF]
        gsem = rest[NBUF:2 * NBUF]
        ssem = rest[2 * NBUF:]
        c = lax.axis_index("c")
        s = lax.axis_index("s")
        r0 = s * rpt
        pltpu.sync_copy(src_hbm.at[s], src_v)
        pltpu.sync_copy(dst_hbm.at[s], dst_v)
        pltpu.sync_copy(ones_hbm.at[pl.ds(r0, rpt), :],
                        deg_sh.at[pl.ds(r0, rpt), :])
        pltpu.sync_copy(ones_hbm.at[pl.ds(0, k), :], onesb)
        plsc.subcore_barrier()

        def body_deg(i, carry):
            pltpu.sync_copy(onesb, deg_sh.at[dst_v.at[i]], add=True)
            return carry

        lax.fori_loop(0, ncw, body_deg, 0)
        plsc.subcore_barrier()
        for t in range(rpt // RB):
            rr = r0 + t * RB
            pltpu.sync_copy(deg_sh.at[pl.ds(rr, RB), :], degb)
            pltpu.sync_copy(xbf.at[c, pl.ds(rr, RB), :], xb)

            def body_rows(r, carry):
                d = degb[r, :]
                di = plsc.bitcast(d, jnp.int32)
                y = plsc.bitcast(jnp.int32(0x5F3759DF) - (di >> 1), jnp.float32)
                h = d * 0.5
                y = y * (1.5 - h * y * y)
                y = y * (1.5 - h * y * y)
                y = y * (1.5 - h * y * y)
                dvb[r, :] = y
                yb = plsc.pack(y, y, format=plsc.PackFormat.INTERLEAVED)
                for q in range(4):
                    ub[r, pl.ds(32 * q, 32)] = xb[r, pl.ds(32 * q, 32)] * yb
                return carry

            lax.fori_loop(0, RB, body_rows, 0)
            pltpu.sync_copy(ub, u0_o.at[c, pl.ds(rr, RB), :])
            pltpu.sync_copy(ub, acc_sh.at[pl.ds(rr, RB), :])

            @pl.when(c == 0)
            def _():
                pltpu.sync_copy(dvb, dinv_o.at[pl.ds(rr, RB), :])

        plsc.subcore_barrier()
        uj = u0_o.at[c]

        def start_g(i, b):
            pltpu.async_copy(uj.at[src_v.at[i]], bufs[b], gsem[b])

        def wait_g(b):
            pltpu.make_async_copy(uj.at[src_v.at[0]], bufs[b], gsem[b]).wait()

        for b in range(NBUF):
            start_g(b, b)

        def body(i, carry):
            ei = NBUF * i
            for b in range(NBUF):
                wait_g(b)
                pltpu.async_copy(bufs[b], acc_sh.at[dst_v.at[ei + b]],
                                 ssem[b], add=True)
            for b in range(NBUF):
                pltpu.make_async_copy(bufs[b], acc_sh.at[dst_v.at[0]],
                                      ssem[b]).wait()
                start_g(jnp.minimum(ei + b + NBUF, ncw - 1), b)
            return carry

        lax.fori_loop(0, ncw // NBUF, body, 0)
        for b in range(NBUF):
            wait_g(b)
        plsc.subcore_barrier()
        pltpu.sync_copy(acc_sh.at[pl.ds(r0, rpt), :],
                        acc1_o.at[c, pl.ds(r0, rpt), :])

    return p1


def _xcast(x, np_, bm):
    def body(x_ref, out_ref):
        xb = x_ref[...]
        out_ref[0] = xb[:, 0:128].astype(jnp.bfloat16)
        out_ref[1] = xb[:, 128:256].astype(jnp.bfloat16)

    return pl.pallas_call(
        body,
        grid=(np_ // bm,),
        in_specs=[pl.BlockSpec((bm, 256), lambda i: (i, 0))],
        out_specs=pl.BlockSpec((2, bm, 128), lambda i: (0, i, 0)),
        out_shape=jax.ShapeDtypeStruct((2, np_, 128), jnp.bfloat16),
    )(x)


def _t_layer(acc, dinv16, w, b, np_, bm, nb_in, nb_out):
    kdim = nb_in * 128
    hdim = nb_out * 128

    def body(acc_ref, dinv_ref, w_ref, b_ref, out_ref):
        a = acc_ref[...].astype(jnp.float32)
        dv = dinv_ref[...][:, 0:1]
        wm = w_ref[...]
        h = jnp.zeros((bm, hdim), jnp.float32)
        for j in range(nb_in):
            h = h + jnp.dot(a[j] * dv, wm[j * 128:(j + 1) * 128, :],
                            preferred_element_type=jnp.float32)
        h = jnp.maximum(h + b_ref[...], 0.0)
        u = (h * dv).astype(jnp.bfloat16)
        for j in range(nb_out):
            out_ref[j] = u[:, j * 128:(j + 1) * 128]

    return pl.pallas_call(
        body,
        grid=(np_ // bm,),
        in_specs=[
            pl.BlockSpec((nb_in, bm, 128), lambda i: (0, i, 0)),
            pl.BlockSpec((bm, 16), lambda i: (i, 0)),
            pl.BlockSpec((kdim, hdim), lambda i: (0, 0)),
            pl.BlockSpec((1, hdim), lambda i: (0, 0)),
        ],
        out_specs=pl.BlockSpec((nb_out, bm, 128), lambda i: (0, i, 0)),
        out_shape=jax.ShapeDtypeStruct((nb_out, np_, 128), jnp.bfloat16),
    )(acc, dinv16, w, b)


def _t2(acc, dinv16, w2, b2, w3, wl16, np_, bm):
    def body(acc_ref, dinv_ref, w_ref, b_ref, w3_ref, wl_ref, out_ref):
        a = acc_ref[...].astype(jnp.float32)
        dv = dinv_ref[...][:, 0:1]
        wm = w_ref[...]
        m3 = jnp.dot(w3_ref[...], wl_ref[...], preferred_element_type=jnp.float32)
        h = jnp.zeros((bm, 512), jnp.float32)
        for j in range(4):
            h = h + jnp.dot(a[j] * dv, wm[j * 128:(j + 1) * 128, :],
                            preferred_element_type=jnp.float32)
        h = jnp.maximum(h + b_ref[...], 0.0)
        z = jnp.dot(h, m3, preferred_element_type=jnp.float32)
        out_ref[...] = z * dv

    return pl.pallas_call(
        body,
        grid=(np_ // bm,),
        in_specs=[
            pl.BlockSpec((4, bm, 128), lambda i: (0, i, 0)),
            pl.BlockSpec((bm, 16), lambda i: (i, 0)),
            pl.BlockSpec((512, 512), lambda i: (0, 0)),
            pl.BlockSpec((1, 512), lambda i: (0, 0)),
            pl.BlockSpec((512, 512), lambda i: (0, 0)),
            pl.BlockSpec((512, 16), lambda i: (0, 0)),
        ],
        out_specs=pl.BlockSpec((bm, 16), lambda i: (i, 0)),
        out_shape=jax.ShapeDtypeStruct((np_, 16), jnp.float32),
    )(acc, dinv16, w2, b2, w3, wl16)


def _t3(accp, uz, dinv16, batch2, b3, wl16, bl16, np_, g):
    def body(accp_ref, uz_ref, dinv_ref, b_ref, b3_ref, wl_ref, bl_ref, out_ref):
        a = accp_ref[...]
        cb = jnp.dot(b3_ref[...], wl_ref[...], preferred_element_type=jnp.float32)
        r3 = (a[0] + a[1] - uz_ref[...]) * dinv_ref[...]
        bb = b_ref[...]
        oh = (bb == lax.broadcasted_iota(jnp.int32, (1, g), 1))
        oh = oh.astype(jnp.float32)
        r3e = jnp.concatenate([r3, jnp.ones_like(r3)], axis=1)
        se = lax.dot_general(oh, r3e, (((0,), (0,)), ((), ())),
                             preferred_element_type=jnp.float32)
        sums = se[:, 0:16]
        cnt = se[:, 16:17]
        pooled = sums / jnp.maximum(cnt, 1.0)
        ind = (cnt > 0.0).astype(jnp.float32)
        out_ref[...] = pooled + ind * cb + bl_ref[...]

    return pl.pallas_call(
        body,
        out_shape=jax.ShapeDtypeStruct((g, 16), jnp.float32),
    )(accp, uz, dinv16, batch2, b3, wl16, bl16)


def kernel(x, edge_index, batch, W1, b1, W2, b2, W3, b3, Wl, bl):
    n, d_in = x.shape
    e = edge_index.shape[1]
    h = W2.shape[0]
    g = 64
    c_out = Wl.shape[1]
    np_ = 10240
    bm = 2048
    kw = 125
    kn = 125

    src = edge_index[0]
    dst = edge_index[1]
    ncw = e // kw // NS
    ncn = e // kn // (NC * NS)
    src_w = src.reshape(NS, ncw, kw)
    dst_w = dst.reshape(NS, ncw, kw)
    src_n = src.reshape(NC * NS, ncn, kn)
    dst_n = dst.reshape(NC * NS, ncn, kn)

    x_p = jnp.pad(x, ((0, np_ - n), (0, 0)))
    batch2 = jnp.pad(batch, (0, np_ - n), constant_values=g).reshape(np_, 1)
    ones16 = jnp.ones((np_, 16), jnp.float32)
    b1r = b1.reshape(1, h)
    b2r = b2.reshape(1, h)
    b3r = b3.reshape(1, h)
    wl16 = jnp.pad(Wl, ((0, 0), (0, 16 - c_out)))
    bl16 = jnp.pad(bl.reshape(1, c_out), ((0, 0), (0, 16 - c_out)))

    prop16 = _sc_propagate(np_, e, 16, 1, kn)
    prop512 = _sc_propagate(np_, e, 128, 4, kw, dtype=jnp.bfloat16)
    layer1 = _sc_layer1(np_, e, kw)

    xbf = _xcast(x_p, np_, bm)
    acc1, dinv16, _u0 = layer1(xbf, ones16, src_w, dst_w)
    u1 = _t_layer(acc1, dinv16, W1, b1r, np_, bm, 2, 4)

    acc2 = prop512(u1, src_w, dst_w)
    uz = _t2(acc2, dinv16, W2, b2r, W3, wl16, np_, bm)

    acc3 = prop16(uz[None], src_n, dst_n)
    out16 = _t3(acc3, uz, dinv16, batch2, b3r, wl16, bl16, np_, g)
    return out16[:g, :c_out]


# speedup vs baseline: 1.0859x; 1.0859x over previous
import functools

import jax
import jax.numpy as jnp
from jax import lax
from jax.experimental import pallas as pl
from jax.experimental.pallas import tpu as pltpu
from jax.experimental.pallas import tpu_sc as plsc

NC = 2
NS = 16


def _sc_propagate(np_, e, w, nb, k, const_u=False, dtype=jnp.float32):
    npass = max(nb // NC, 1)
    out_sc = NC if nb == 1 else nb
    if nb == 1:
        nchunk = e // k // (NC * NS)
    else:
        nchunk = e // k // NS
    NBUF = 4
    assert nchunk % NBUF == 0
    rpt = np_ // NS

    @functools.partial(
        pl.kernel,
        out_type=jax.ShapeDtypeStruct((out_sc, np_, w), dtype),
        mesh=plsc.VectorSubcoreMesh(core_axis_name="c", subcore_axis_name="s"),
        compiler_params=pltpu.CompilerParams(use_tc_tiling_on_sc=False),
        scratch_types=[
            pltpu.VMEM_SHARED((np_, w), dtype),
            pltpu.VMEM((nchunk, k), jnp.int32),
            pltpu.VMEM((nchunk, k), jnp.int32),
        ] + [pltpu.VMEM((k, w), dtype)] * NBUF
          + [pltpu.SemaphoreType.DMA] * (2 * NBUF),
    )
    def prop(u_hbm, src_hbm, dst_hbm, out_hbm, acc_sh, src_v, dst_v, *rest):
        bufs = rest[:NBUF]
        gsem = rest[NBUF:2 * NBUF]
        ssem = rest[2 * NBUF:]
        c = lax.axis_index("c")
        s = lax.axis_index("s")
        r0 = s * rpt
        if nb == 1:
            dsec = c * NS + s
        else:
            dsec = s
        pltpu.sync_copy(dst_hbm.at[dsec], dst_v)

        def start_g(uj, i, b):
            pltpu.async_copy(uj.at[src_v.at[i]], bufs[b], gsem[b])

        def wait_g(uj, b):
            pltpu.make_async_copy(uj.at[src_v.at[0]], bufs[b], gsem[b]).wait()

        def start_s(i, b):
            pltpu.async_copy(bufs[b], acc_sh.at[dst_v.at[i]], ssem[b], add=True)

        def wait_s(b):
            pltpu.make_async_copy(bufs[b], acc_sh.at[dst_v.at[0]], ssem[b]).wait()

        for p in range(npass):
            if nb == 1:
                j = 0
                ssec = dsec
            else:
                j = p * NC + c
                ssec = s
            uj = u_hbm.at[j]
            if not const_u and p == 0:
                pltpu.sync_copy(src_hbm.at[ssec], src_v)
            pltpu.sync_copy(u_hbm.at[j, pl.ds(r0, rpt), :],
                            acc_sh.at[pl.ds(r0, rpt), :])
            if const_u and p == 0:
                pltpu.sync_copy(u_hbm.at[0, pl.ds(0, k), :], bufs[0])
            plsc.subcore_barrier()

            if const_u:
                def body_c(i, carry):
                    pltpu.sync_copy(bufs[0], acc_sh.at[dst_v.at[i]], add=True)
                    return carry

                lax.fori_loop(0, nchunk, body_c, 0)
            else:
                for b in range(NBUF):
                    start_g(uj, b, b)

                def body(i, carry):
                    ei = NBUF * i
                    for b in range(NBUF):
                        wait_g(uj, b)
                        start_s(ei + b, b)
                    for b in range(NBUF):
                        wait_s(b)
                        start_g(uj, jnp.minimum(ei + b + NBUF, nchunk - 1), b)
                    return carry

                lax.fori_loop(0, nchunk // NBUF, body, 0)
                for b in range(NBUF):
                    wait_g(uj, b)
            plsc.subcore_barrier()
            oj = c if nb == 1 else j
            pltpu.sync_copy(acc_sh.at[pl.ds(r0, rpt), :],
                            out_hbm.at[oj, pl.ds(r0, rpt), :])
            if p + 1 < npass:
                plsc.subcore_barrier()

    return prop


def _t0(degp, x, np_, bm):
    d_in = x.shape[1]

    def body(degp_ref, x_ref, dinv16_ref, u0_ref):
        d = degp_ref[...]
        deg = d[0, :, 0:1] + d[1, :, 0:1] - 1.0
        dinv = lax.rsqrt(deg)
        dinv16_ref[...] = jnp.broadcast_to(dinv, (bm, 16))
        xb = x_ref[...]
        u0_ref[0] = (xb[:, 0:128] * dinv).astype(jnp.bfloat16)
        u0_ref[1] = (xb[:, 128:256] * dinv).astype(jnp.bfloat16)

    return pl.pallas_call(
        body,
        grid=(np_ // bm,),
        in_specs=[
            pl.BlockSpec((2, bm, 16), lambda i: (0, i, 0)),
            pl.BlockSpec((bm, d_in), lambda i: (i, 0)),
        ],
        out_specs=[
            pl.BlockSpec((bm, 16), lambda i: (i, 0)),
            pl.BlockSpec((2, bm, 128), lambda i: (0, i, 0)),
        ],
        out_shape=[
            jax.ShapeDtypeStruct((np_, 16), jnp.float32),
            jax.ShapeDtypeStruct((2, np_, 128), jnp.bfloat16),
        ],
    )(degp, x)


def _t_layer(acc, dinv16, w, b, np_, bm, nb_in, nb_out):
    kdim = nb_in * 128
    hdim = nb_out * 128

    def body(acc_ref, dinv_ref, w_ref, b_ref, out_ref):
        a = acc_ref[...].astype(jnp.float32)
        dv = dinv_ref[...][:, 0:1]
        wm = w_ref[...]
        h = jnp.zeros((bm, hdim), jnp.float32)
        for j in range(nb_in):
            h = h + jnp.dot(a[j] * dv, wm[j * 128:(j + 1) * 128, :],
                            preferred_element_type=jnp.float32)
        h = jnp.maximum(h + b_ref[...], 0.0)
        u = (h * dv).astype(jnp.bfloat16)
        for j in range(nb_out):
            out_ref[j] = u[:, j * 128:(j + 1) * 128]

    return pl.pallas_call(
        body,
        grid=(np_ // bm,),
        in_specs=[
            pl.BlockSpec((nb_in, bm, 128), lambda i: (0, i, 0)),
            pl.BlockSpec((bm, 16), lambda i: (i, 0)),
            pl.BlockSpec((kdim, hdim), lambda i: (0, 0)),
            pl.BlockSpec((1, hdim), lambda i: (0, 0)),
        ],
        out_specs=pl.BlockSpec((nb_out, bm, 128), lambda i: (0, i, 0)),
        out_shape=jax.ShapeDtypeStruct((nb_out, np_, 128), jnp.bfloat16),
    )(acc, dinv16, w, b)


def _t2(acc, dinv16, w2, b2, w3, wl16, np_, bm):
    def body(acc_ref, dinv_ref, w_ref, b_ref, w3_ref, wl_ref, out_ref):
        a = acc_ref[...].astype(jnp.float32)
        dv = dinv_ref[...][:, 0:1]
        wm = w_ref[...]
        m3 = jnp.dot(w3_ref[...], wl_ref[...], preferred_element_type=jnp.float32)
        h = jnp.zeros((bm, 512), jnp.float32)
        for j in range(4):
            h = h + jnp.dot(a[j] * dv, wm[j * 128:(j + 1) * 128, :],
                            preferred_element_type=jnp.float32)
        h = jnp.maximum(h + b_ref[...], 0.0)
        z = jnp.dot(h, m3, preferred_element_type=jnp.float32)
        out_ref[...] = z * dv

    return pl.pallas_call(
        body,
        grid=(np_ // bm,),
        in_specs=[
            pl.BlockSpec((4, bm, 128), lambda i: (0, i, 0)),
            pl.BlockSpec((bm, 16), lambda i: (i, 0)),
            pl.BlockSpec((512, 512), lambda i: (0, 0)),
            pl.BlockSpec((1, 512), lambda i: (0, 0)),
            pl.BlockSpec((512, 512), lambda i: (0, 0)),
            pl.BlockSpec((512, 16), lambda i: (0, 0)),
        ],
        out_specs=pl.BlockSpec((bm, 16), lambda i: (i, 0)),
        out_shape=jax.ShapeDtypeStruct((np_, 16), jnp.float32),
    )(acc, dinv16, w2, b2, w3, wl16)


def _t3(accp, uz, dinv16, batch2, b3, wl16, bl16, np_, g):
    def body(accp_ref, uz_ref, dinv_ref, b_ref, b3_ref, wl_ref, bl_ref, out_ref):
        a = accp_ref[...]
        cb = jnp.dot(b3_ref[...], wl_ref[...], preferred_element_type=jnp.float32)
        r3 = (a[0] + a[1] - uz_ref[...]) * dinv_ref[...]
        bb = b_ref[...]
        oh = (bb == lax.broadcasted_iota(jnp.int32, (1, g), 1))
        oh = oh.astype(jnp.float32)
        r3e = jnp.concatenate([r3, jnp.ones_like(r3)], axis=1)
        se = lax.dot_general(oh, r3e, (((0,), (0,)), ((), ())),
                             preferred_element_type=jnp.float32)
        sums = se[:, 0:16]
        cnt = se[:, 16:17]
        pooled = sums / jnp.maximum(cnt, 1.0)
        ind = (cnt > 0.0).astype(jnp.float32)
        out_ref[...] = pooled + ind * cb + bl_ref[...]

    return pl.pallas_call(
        body,
        out_shape=jax.ShapeDtypeStruct((g, 16), jnp.float32),
    )(accp, uz, dinv16, batch2, b3, wl16, bl16)


def kernel(x, edge_index, batch, W1, b1, W2, b2, W3, b3, Wl, bl):
    n, d_in = x.shape
    e = edge_index.shape[1]
    h = W2.shape[0]
    g = 64
    c_out = Wl.shape[1]
    np_ = 10240
    bm = 2048
    kw = 125
    kn = 125

    src = edge_index[0]
    dst = edge_index[1]
    ncw = e // kw // NS
    ncn = e // kn // (NC * NS)
    src_w = src.reshape(NS, ncw, kw)
    dst_w = dst.reshape(NS, ncw, kw)
    src_n = src.reshape(NC * NS, ncn, kn)
    dst_n = dst.reshape(NC * NS, ncn, kn)

    x_p = jnp.pad(x, ((0, np_ - n), (0, 0)))
    batch2 = jnp.pad(batch, (0, np_ - n), constant_values=g).reshape(np_, 1)
    ones16 = jnp.ones((np_, 16), jnp.float32)
    b1r = b1.reshape(1, h)
    b2r = b2.reshape(1, h)
    b3r = b3.reshape(1, h)
    wl16 = jnp.pad(Wl, ((0, 0), (0, 16 - c_out)))
    bl16 = jnp.pad(bl.reshape(1, c_out), ((0, 0), (0, 16 - c_out)))

    propdeg = _sc_propagate(np_, e, 16, 1, kn, const_u=True)
    prop16 = _sc_propagate(np_, e, 16, 1, kn)
    prop256 = _sc_propagate(np_, e, 128, 2, kw, dtype=jnp.bfloat16)
    prop512 = _sc_propagate(np_, e, 128, 4, kw, dtype=jnp.bfloat16)

    degp = propdeg(ones16[None], src_n, dst_n)
    dinv16, u0 = _t0(degp, x_p, np_, bm)

    acc1 = prop256(u0, src_w, dst_w)
    u1 = _t_layer(acc1, dinv16, W1, b1r, np_, bm, 2, 4)

    acc2 = prop512(u1, src_w, dst_w)
    uz = _t2(acc2, dinv16, W2, b2r, W3, wl16, np_, bm)

    acc3 = prop16(uz[None], src_n, dst_n)
    out16 = _t3(acc3, uz, dinv16, batch2, b3r, wl16, bl16, np_, g)
    return out16[:g, :c_out]


# speedup vs baseline: 1.0882x; 1.0022x over previous
import functools

import jax
import jax.numpy as jnp
from jax import lax
from jax.experimental import pallas as pl
from jax.experimental.pallas import tpu as pltpu
from jax.experimental.pallas import tpu_sc as plsc

NC = 2
NS = 16


def _sc_propagate(np_, e, w, nb, k, const_u=False, dtype=jnp.float32):
    npass = max(nb // NC, 1)
    out_sc = NC if nb == 1 else nb
    if nb == 1:
        nchunk = e // k // (NC * NS)
    else:
        nchunk = e // k // NS
    NBUF = 4
    assert nchunk % NBUF == 0
    rpt = np_ // NS

    @functools.partial(
        pl.kernel,
        out_type=jax.ShapeDtypeStruct((out_sc, np_, w), dtype),
        mesh=plsc.VectorSubcoreMesh(core_axis_name="c", subcore_axis_name="s"),
        compiler_params=pltpu.CompilerParams(use_tc_tiling_on_sc=False),
        scratch_types=[
            pltpu.VMEM_SHARED((np_, w), dtype),
            pltpu.VMEM((nchunk, k), jnp.int32),
            pltpu.VMEM((nchunk, k), jnp.int32),
        ] + [pltpu.VMEM((k, w), dtype)] * NBUF
          + [pltpu.SemaphoreType.DMA] * (2 * NBUF),
    )
    def prop(u_hbm, src_hbm, dst_hbm, out_hbm, acc_sh, src_v, dst_v, *rest):
        bufs = rest[:NBUF]
        gsem = rest[NBUF:2 * NBUF]
        ssem = rest[2 * NBUF:]
        c = lax.axis_index("c")
        s = lax.axis_index("s")
        r0 = s * rpt
        if nb == 1:
            dsec = c * NS + s
        else:
            dsec = s
        pltpu.sync_copy(dst_hbm.at[dsec], dst_v)

        def start_g(uj, i, b):
            pltpu.async_copy(uj.at[src_v.at[i]], bufs[b], gsem[b])

        def wait_g(uj, b):
            pltpu.make_async_copy(uj.at[src_v.at[0]], bufs[b], gsem[b]).wait()

        def start_s(i, b):
            pltpu.async_copy(bufs[b], acc_sh.at[dst_v.at[i]], ssem[b], add=True)

        def wait_s(b):
            pltpu.make_async_copy(bufs[b], acc_sh.at[dst_v.at[0]], ssem[b]).wait()

        for p in range(npass):
            if nb == 1:
                j = 0
                ssec = dsec
            else:
                j = p * NC + c
                ssec = s
            uj = u_hbm.at[j]
            if not const_u and p == 0:
                pltpu.sync_copy(src_hbm.at[ssec], src_v)
            pltpu.sync_copy(u_hbm.at[j, pl.ds(r0, rpt), :],
                            acc_sh.at[pl.ds(r0, rpt), :])
            if const_u and p == 0:
                pltpu.sync_copy(u_hbm.at[0, pl.ds(0, k), :], bufs[0])
            plsc.subcore_barrier()

            if const_u:
                def body_c(i, carry):
                    pltpu.async_copy(bufs[0], acc_sh.at[dst_v.at[i]],
                                     ssem[0], add=True)
                    return carry

                lax.fori_loop(0, nchunk, body_c, 0)

                def body_w(i, carry):
                    pltpu.make_async_copy(bufs[0], acc_sh.at[dst_v.at[0]],
                                          ssem[0]).wait()
                    return carry

                lax.fori_loop(0, nchunk, body_w, 0)
            else:
                for b in range(NBUF):
                    start_g(uj, b, b)

                def body(i, carry):
                    ei = NBUF * i
                    for b in range(NBUF):
                        wait_g(uj, b)
                        start_s(ei + b, b)
                    for b in range(NBUF):
                        wait_s(b)
                        start_g(uj, jnp.minimum(ei + b + NBUF, nchunk - 1), b)
                    return carry

                lax.fori_loop(0, nchunk // NBUF, body, 0)
                for b in range(NBUF):
                    wait_g(uj, b)
            plsc.subcore_barrier()
            oj = c if nb == 1 else j
            pltpu.sync_copy(acc_sh.at[pl.ds(r0, rpt), :],
                            out_hbm.at[oj, pl.ds(r0, rpt), :])
            if p + 1 < npass:
                plsc.subcore_barrier()

    return prop


def _t0(degp, x, np_, bm):
    d_in = x.shape[1]

    def body(degp_ref, x_ref, dinv16_ref, u0_ref):
        d = degp_ref[...]
        deg = d[0, :, 0:1] + d[1, :, 0:1] - 1.0
        dinv = lax.rsqrt(deg)
        dinv16_ref[...] = jnp.broadcast_to(dinv, (bm, 16))
        xb = x_ref[...]
        u0_ref[0] = (xb[:, 0:128] * dinv).astype(jnp.bfloat16)
        u0_ref[1] = (xb[:, 128:256] * dinv).astype(jnp.bfloat16)

    return pl.pallas_call(
        body,
        grid=(np_ // bm,),
        in_specs=[
            pl.BlockSpec((2, bm, 16), lambda i: (0, i, 0)),
            pl.BlockSpec((bm, d_in), lambda i: (i, 0)),
        ],
        out_specs=[
            pl.BlockSpec((bm, 16), lambda i: (i, 0)),
            pl.BlockSpec((2, bm, 128), lambda i: (0, i, 0)),
        ],
        out_shape=[
            jax.ShapeDtypeStruct((np_, 16), jnp.float32),
            jax.ShapeDtypeStruct((2, np_, 128), jnp.bfloat16),
        ],
    )(degp, x)


def _t_layer(acc, dinv16, w, b, np_, bm, nb_in, nb_out):
    kdim = nb_in * 128
    hdim = nb_out * 128

    def body(acc_ref, dinv_ref, w_ref, b_ref, out_ref):
        a = acc_ref[...].astype(jnp.float32)
        dv = dinv_ref[...][:, 0:1]
        wm = w_ref[...]
        h = jnp.zeros((bm, hdim), jnp.float32)
        for j in range(nb_in):
            h = h + jnp.dot(a[j] * dv, wm[j * 128:(j + 1) * 128, :],
                            preferred_element_type=jnp.float32)
        h = jnp.maximum(h + b_ref[...], 0.0)
        u = (h * dv).astype(jnp.bfloat16)
        for j in range(nb_out):
            out_ref[j] = u[:, j * 128:(j + 1) * 128]

    return pl.pallas_call(
        body,
        grid=(np_ // bm,),
        in_specs=[
            pl.BlockSpec((nb_in, bm, 128), lambda i: (0, i, 0)),
            pl.BlockSpec((bm, 16), lambda i: (i, 0)),
            pl.BlockSpec((kdim, hdim), lambda i: (0, 0)),
            pl.BlockSpec((1, hdim), lambda i: (0, 0)),
        ],
        out_specs=pl.BlockSpec((nb_out, bm, 128), lambda i: (0, i, 0)),
        out_shape=jax.ShapeDtypeStruct((nb_out, np_, 128), jnp.bfloat16),
    )(acc, dinv16, w, b)


def _t2(acc, dinv16, w2, b2, w3, wl16, np_, bm):
    def body(acc_ref, dinv_ref, w_ref, b_ref, w3_ref, wl_ref, out_ref):
        a = acc_ref[...].astype(jnp.float32)
        dv = dinv_ref[...][:, 0:1]
        wm = w_ref[...]
        m3 = jnp.dot(w3_ref[...], wl_ref[...], preferred_element_type=jnp.float32)
        h = jnp.zeros((bm, 512), jnp.float32)
        for j in range(4):
            h = h + jnp.dot(a[j] * dv, wm[j * 128:(j + 1) * 128, :],
                            preferred_element_type=jnp.float32)
        h = jnp.maximum(h + b_ref[...], 0.0)
        z = jnp.dot(h, m3, preferred_element_type=jnp.float32)
        out_ref[...] = z * dv

    return pl.pallas_call(
        body,
        grid=(np_ // bm,),
        in_specs=[
            pl.BlockSpec((4, bm, 128), lambda i: (0, i, 0)),
            pl.BlockSpec((bm, 16), lambda i: (i, 0)),
            pl.BlockSpec((512, 512), lambda i: (0, 0)),
            pl.BlockSpec((1, 512), lambda i: (0, 0)),
            pl.BlockSpec((512, 512), lambda i: (0, 0)),
            pl.BlockSpec((512, 16), lambda i: (0, 0)),
        ],
        out_specs=pl.BlockSpec((bm, 16), lambda i: (i, 0)),
        out_shape=jax.ShapeDtypeStruct((np_, 16), jnp.float32),
    )(acc, dinv16, w2, b2, w3, wl16)


def _t3(accp, uz, dinv16, batch2, b3, wl16, bl16, np_, g):
    def body(accp_ref, uz_ref, dinv_ref, b_ref, b3_ref, wl_ref, bl_ref, out_ref):
        a = accp_ref[...]
        cb = jnp.dot(b3_ref[...], wl_ref[...], preferred_element_type=jnp.float32)
        r3 = (a[0] + a[1] - uz_ref[...]) * dinv_ref[...]
        bb = b_ref[...]
        oh = (bb == lax.broadcasted_iota(jnp.int32, (1, g), 1))
        oh = oh.astype(jnp.float32)
        r3e = jnp.concatenate([r3, jnp.ones_like(r3)], axis=1)
        se = lax.dot_general(oh, r3e, (((0,), (0,)), ((), ())),
                             preferred_element_type=jnp.float32)
        sums = se[:, 0:16]
        cnt = se[:, 16:17]
        pooled = sums / jnp.maximum(cnt, 1.0)
        ind = (cnt > 0.0).astype(jnp.float32)
        out_ref[...] = pooled + ind * cb + bl_ref[...]

    return pl.pallas_call(
        body,
        out_shape=jax.ShapeDtypeStruct((g, 16), jnp.float32),
    )(accp, uz, dinv16, batch2, b3, wl16, bl16)


def kernel(x, edge_index, batch, W1, b1, W2, b2, W3, b3, Wl, bl):
    n, d_in = x.shape
    e = edge_index.shape[1]
    h = W2.shape[0]
    g = 64
    c_out = Wl.shape[1]
    np_ = 10240
    bm = 5120
    kw = 125
    kn = 125

    src = edge_index[0]
    dst = edge_index[1]
    ncw = e // kw // NS
    ncn = e // kn // (NC * NS)
    src_w = src.reshape(NS, ncw, kw)
    dst_w = dst.reshape(NS, ncw, kw)
    src_n = src.reshape(NC * NS, ncn, kn)
    dst_n = dst.reshape(NC * NS, ncn, kn)

    x_p = jnp.pad(x, ((0, np_ - n), (0, 0)))
    batch2 = jnp.pad(batch, (0, np_ - n), constant_values=g).reshape(np_, 1)
    ones16 = jnp.ones((np_, 16), jnp.float32)
    b1r = b1.reshape(1, h)
    b2r = b2.reshape(1, h)
    b3r = b3.reshape(1, h)
    wl16 = jnp.pad(Wl, ((0, 0), (0, 16 - c_out)))
    bl16 = jnp.pad(bl.reshape(1, c_out), ((0, 0), (0, 16 - c_out)))

    propdeg = _sc_propagate(np_, e, 16, 1, kn, const_u=True)
    prop16 = _sc_propagate(np_, e, 16, 1, kn)
    prop256 = _sc_propagate(np_, e, 128, 2, kw, dtype=jnp.bfloat16)
    prop512 = _sc_propagate(np_, e, 128, 4, kw, dtype=jnp.bfloat16)

    degp = propdeg(ones16[None], src_n, dst_n)
    dinv16, u0 = _t0(degp, x_p, np_, bm)

    acc1 = prop256(u0, src_w, dst_w)
    u1 = _t_layer(acc1, dinv16, W1, b1r, np_, bm, 2, 4)

    acc2 = prop512(u1, src_w, dst_w)
    uz = _t2(acc2, dinv16, W2, b2r, W3, wl16, np_, bm)

    acc3 = prop16(uz[None], src_n, dst_n)
    out16 = _t3(acc3, uz, dinv16, batch2, b3r, wl16, bl16, np_, g)
    return out16[:g, :c_out]
